# Initial kernel scaffold; baseline (speedup 1.0000x reference)
import jax
import jax.numpy as jnp
from jax.experimental import pallas as pl


def kernel(x, edge_index, W_lin, b_lin, W_self, b_self, bias):
    raise NotImplementedError("write your pallas kernel here")


# speedup vs baseline: 12.8583x; 12.8583x over previous
import functools

import jax
import jax.numpy as jnp
from jax import lax
from jax.experimental import pallas as pl
from jax.experimental.pallas import tpu as pltpu
from jax.experimental.pallas import tpu_sc as plsc

N_NODES = 10000
N_EDGES = 320000
D = 128

NC = 2
NS = 16
NW = NC * NS

CHUNK = 128
CH = 80
EP = NW * CH * CHUNK
ACC_ROWS = 10240
RPT = ACC_ROWS // NS


CHH = CH // 2


def _sc_agg_body(src_hbm, dst_hbm, x_hbm, out_hbm,
                 src_v, dst_v, rows_v, acc_sh, gsem0, gsem1):
    c = lax.axis_index("c")
    s = lax.axis_index("s")
    wid = s * NC + c

    def _zero_row(i, carry):
        for j in range(D // 16):
            rows_v[0, i, pl.ds(j * 16, 16)] = jnp.zeros((16,), jnp.float32)
        return carry

    lax.fori_loop(0, 128, _zero_row, 0)
    for k in range(RPT // 128):
        pltpu.sync_copy(rows_v.at[0], acc_sh.at[pl.ds(s * RPT + k * 128, 128)])
    plsc.subcore_barrier()

    sems = (gsem0, gsem1)

    for half in range(2):
        base = wid * CH + half * CHH
        pltpu.sync_copy(src_hbm.at[pl.ds(base, CHH)], src_v)
        pltpu.sync_copy(dst_hbm.at[pl.ds(base, CHH)], dst_v)

        for b in range(2):
            pltpu.async_copy(x_hbm.at[src_v.at[b]], rows_v.at[b], sems[b])

        def _body(i, carry):
            g = i * 2
            for b in range(2):
                j = g + b
                pltpu.make_async_copy(x_hbm.at[src_v.at[j]], rows_v.at[b],
                                      sems[b]).wait()
                pltpu.sync_copy(rows_v.at[b], acc_sh.at[dst_v.at[j]],
                                add=True)
                nxt = j + 2

                @pl.when(nxt < CHH)
                def _():
                    pltpu.async_copy(x_hbm.at[src_v.at[nxt]], rows_v.at[b],
                                     sems[b])
            return carry

        lax.fori_loop(0, CHH // 2, _body, 0)
    plsc.subcore_barrier()

    for k in range(RPT // 128):
        r0 = s * RPT + k * 128
        pltpu.sync_copy(acc_sh.at[pl.ds(r0, 128)], out_hbm.at[c, pl.ds(r0, 128)])


_sc_agg = functools.partial(
    pl.kernel,
    mesh=plsc.VectorSubcoreMesh(core_axis_name="c", subcore_axis_name="s"),
    out_type=jax.ShapeDtypeStruct((NC, ACC_ROWS, D), jnp.float32),
    scratch_types=[
        pltpu.VMEM((CH // 2, CHUNK), jnp.int32),
        pltpu.VMEM((CH // 2, CHUNK), jnp.int32),
        pltpu.VMEM((2, CHUNK, D), jnp.float32),
        pltpu.VMEM_SHARED((ACC_ROWS, D), jnp.float32),
        pltpu.SemaphoreType.DMA,
        pltpu.SemaphoreType.DMA,
    ],
)(_sc_agg_body)


def _tc_body(h0_ref, h1_ref, x_ref, wl_ref, ws_ref, bl_ref, bs_ref, bb_ref,
             o_ref):
    hsum = h0_ref[0] + h1_ref[0]
    acc = lax.dot_general(hsum, wl_ref[...], (((1,), (1,)), ((), ())),
                          preferred_element_type=jnp.float32)
    acc = acc + lax.dot_general(x_ref[...], ws_ref[...],
                                (((1,), (1,)), ((), ())),
                                preferred_element_type=jnp.float32)
    o_ref[...] = acc + (bl_ref[...] + bs_ref[...] + bb_ref[...])


def _tc_linear(h, x, W_lin, W_self, b_lin, b_self, bias):
    blk = 1000
    return pl.pallas_call(
        _tc_body,
        grid=(N_NODES // blk,),
        in_specs=[
            pl.BlockSpec((1, blk, D), lambda i: (0, i, 0)),
            pl.BlockSpec((1, blk, D), lambda i: (1, i, 0)),
            pl.BlockSpec((blk, D), lambda i: (i, 0)),
            pl.BlockSpec((D, D), lambda i: (0, 0)),
            pl.BlockSpec((D, D), lambda i: (0, 0)),
            pl.BlockSpec((1, D), lambda i: (0, 0)),
            pl.BlockSpec((1, D), lambda i: (0, 0)),
            pl.BlockSpec((1, D), lambda i: (0, 0)),
        ],
        out_specs=pl.BlockSpec((blk, D), lambda i: (i, 0)),
        out_shape=jax.ShapeDtypeStruct((N_NODES, D), jnp.float32),
    )(h, h, x, W_lin, W_self,
      b_lin.reshape(1, D), b_self.reshape(1, D), bias.reshape(1, D))


def kernel(x, edge_index, W_lin, b_lin, W_self, b_self, bias):
    ei = edge_index.astype(jnp.int32)
    src = ei[0]
    dst = ei[1]
    pad_n = EP - N_EDGES
    pad_pos = jnp.arange(pad_n, dtype=jnp.int32)
    pad_src = pad_pos % N_NODES
    pad_dst = N_NODES + pad_pos % (ACC_ROWS - N_NODES)
    src_p = jnp.concatenate([src, pad_src]).reshape(EP // CHUNK, CHUNK)
    dst_p = jnp.concatenate([dst, pad_dst]).reshape(EP // CHUNK, CHUNK)
    h = _sc_agg(src_p, dst_p, x)
    return _tc_linear(h, x, W_lin, W_self, b_lin, b_self, bias)
